# baseline (speedup 1.0000x reference)
import jax
import jax.numpy as jnp
from jax.experimental import pallas as pl

_ADV_W = 0.6
_NORM_W = 0.05
_DATA_W = 0.25


def _loss_kernel(d_ref, fake_ref, real_ref, out_ref):
    n_slice = 4 * 2048 * 3
    adv = -jnp.sum(jnp.log(d_ref[...] + 1e-08)) / d_ref.size
    diff = real_ref[...] - fake_ref[...]
    ch = jax.lax.broadcasted_iota(jnp.int32, diff.shape, 0) % 6
    w_sqrt = jnp.where(ch < 3, (_DATA_W / n_slice) ** 0.5,
                       (_NORM_W / n_slice) ** 0.5)
    t = diff * w_sqrt
    out_ref[...] = jnp.reshape(_ADV_W * adv + jnp.sum(t * t), (1, 1))


def kernel(D_output_fake, fake_data, real_data):
    fake_t = jnp.reshape(jnp.transpose(fake_data, (0, 2, 1)), (24, 2048))
    real_t = jnp.reshape(jnp.transpose(real_data, (0, 2, 1)), (24, 2048))
    out = pl.pallas_call(
        _loss_kernel,
        out_shape=jax.ShapeDtypeStruct((1, 1), jnp.float32),
    )(D_output_fake, fake_t, real_t)
    return out[0, 0]


# speedup vs baseline: 1.0350x; 1.0350x over previous
import jax
import jax.numpy as jnp
from jax.experimental import pallas as pl

_ADV_W = 0.6
_NORM_W = 0.05
_DATA_W = 0.25


def _loss_kernel(d_ref, fake_ref, real_ref, out_ref):
    n_slice = 4 * 2048 * 3
    adv = -jnp.sum(jnp.log(d_ref[...] + 1e-08)) / d_ref.size
    diff = real_ref[...] - fake_ref[...]
    ch = jax.lax.broadcasted_iota(jnp.int32, diff.shape, 1)
    w_sqrt = jnp.where(ch < 3, (_DATA_W / n_slice) ** 0.5,
                       (_NORM_W / n_slice) ** 0.5)
    t = diff * w_sqrt
    out_ref[...] = jnp.reshape(_ADV_W * adv + jnp.sum(t * t), (1, 1))


def kernel(D_output_fake, fake_data, real_data):
    fake_t = jnp.transpose(fake_data, (0, 2, 1))
    real_t = jnp.transpose(real_data, (0, 2, 1))
    out = pl.pallas_call(
        _loss_kernel,
        out_shape=jax.ShapeDtypeStruct((1, 1), jnp.float32),
    )(D_output_fake, fake_t, real_t)
    return out[0, 0]


# speedup vs baseline: 1.9054x; 1.8409x over previous
import jax
import jax.numpy as jnp
from jax.experimental import pallas as pl
from jax.experimental.pallas import tpu as pltpu
from jax.experimental.pallas import tpu as pltpu

_ADV_W = 0.6
_NORM_W = 0.05
_DATA_W = 0.25
_ROWS = 8
_COLS = (4 * 2048 * 6) // _ROWS


def _loss_kernel(d_ref, fake_hbm, real_hbm, out_ref, fake_v, real_v, sems):
    cp_f = pltpu.make_async_copy(
        fake_hbm.reshape(_ROWS, _COLS), fake_v, sems.at[0])
    cp_r = pltpu.make_async_copy(
        real_hbm.reshape(_ROWS, _COLS), real_v, sems.at[1])
    cp_f.start()
    cp_r.start()
    cp_f.wait()
    cp_r.wait()

    n_slice = 4 * 2048 * 3
    adv = -jnp.sum(jnp.log(d_ref[...] + 1e-08)) / d_ref.size
    diff = real_v[...] - fake_v[...]
    lane = jax.lax.broadcasted_iota(jnp.int32, diff.shape, 1)
    w_sqrt = jnp.where(lane % 6 < 3, (_DATA_W / n_slice) ** 0.5,
                       (_NORM_W / n_slice) ** 0.5)
    t = diff * w_sqrt
    out_ref[...] = jnp.reshape(_ADV_W * adv + jnp.sum(t * t), (1, 1))


def kernel(D_output_fake, fake_data, real_data):
    out = pl.pallas_call(
        _loss_kernel,
        in_specs=[
            pl.BlockSpec(memory_space=pltpu.MemorySpace.VMEM),
            pl.BlockSpec(memory_space=pltpu.MemorySpace.HBM),
            pl.BlockSpec(memory_space=pltpu.MemorySpace.HBM),
        ],
        out_specs=pl.BlockSpec(memory_space=pltpu.MemorySpace.VMEM),
        scratch_shapes=[
            pltpu.VMEM((_ROWS, _COLS), jnp.float32),
            pltpu.VMEM((_ROWS, _COLS), jnp.float32),
            pltpu.SemaphoreType.DMA((2,)),
        ],
        out_shape=jax.ShapeDtypeStruct((1, 1), jnp.float32),
    )(D_output_fake, fake_data, real_data)
    return out[0, 0]


# speedup vs baseline: 1.9225x; 1.0090x over previous
import jax
import jax.numpy as jnp
from jax.experimental import pallas as pl
from jax.experimental.pallas import tpu as pltpu

_ADV_W = 0.6
_NORM_W = 0.05
_DATA_W = 0.25


def _loss_kernel(both_ref, out_ref):
    n_slice = 4 * 2048 * 3
    d_vals = both_ref[:, 0:1, 4096:4097]
    adv = -jnp.sum(jnp.log(d_vals + 1e-08)) / 4.0
    diff = both_ref[:, :, 2048:4096] - both_ref[:, :, 0:2048]
    ch = jax.lax.broadcasted_iota(jnp.int32, diff.shape, 1)
    w_sqrt = jnp.where(ch < 3, (_DATA_W / n_slice) ** 0.5,
                       (_NORM_W / n_slice) ** 0.5)
    t = diff * w_sqrt
    out_ref[...] = jnp.reshape(_ADV_W * adv + jnp.sum(t * t), (1, 1))


def kernel(D_output_fake, fake_data, real_data):
    fake_t = jnp.transpose(fake_data, (0, 2, 1))
    real_t = jnp.transpose(real_data, (0, 2, 1))
    d_strip = jnp.broadcast_to(D_output_fake[:, :, None], (4, 6, 128))
    both = jnp.concatenate([fake_t, real_t, d_strip], axis=2)
    both = pltpu.with_memory_space_constraint(both, pltpu.MemorySpace.VMEM)
    out = pl.pallas_call(
        _loss_kernel,
        in_specs=[pl.BlockSpec(memory_space=pltpu.MemorySpace.VMEM)],
        out_specs=pl.BlockSpec(memory_space=pltpu.MemorySpace.VMEM),
        out_shape=jax.ShapeDtypeStruct((1, 1), jnp.float32),
    )(both)
    return out[0, 0]
